# baseline (speedup 1.0000x reference)
import functools
import math

import jax
import jax.numpy as jnp
from jax import lax
from jax.experimental import pallas as pl
from jax.experimental.pallas import tpu as pltpu
from jax.experimental.pallas import tpu_sc as plsc

N_CLS = 32000
PAD = 0
EPS = 0.1 / (N_CLS - 2)
CONF = 0.9
C0 = (N_CLS - 2) * EPS * math.log(EPS) + CONF * math.log(CONF)

RBLK = 128

_INFO = plsc.get_sparse_core_info()
_NC, _NS, _L = _INFO.num_cores, _INFO.num_subcores, _INFO.num_lanes
_NW = _NC * _NS


def _sc_gather_build(n_tok):
    rpw = n_tok // _NW
    mesh = plsc.VectorSubcoreMesh(core_axis_name="c", subcore_axis_name="s")

    @functools.partial(
        pl.kernel, mesh=mesh,
        out_type=jax.ShapeDtypeStruct((_NW, rpw), jnp.float32),
        scratch_types=[
            pltpu.VMEM((rpw,), jnp.int32),
            pltpu.VMEM((rpw,), jnp.int32),
            pltpu.VMEM((rpw,), jnp.float32),
            pltpu.SemaphoreType.DMA,
        ],
    )
    def sc_gather(x_flat_hbm, tgt_hbm, out_hbm, tgt_v, idx_v, g_v, sem):
        wid = lax.axis_index("s") * _NC + lax.axis_index("c")
        base = wid * rpw
        pltpu.sync_copy(tgt_hbm.at[pl.ds(base, rpw)], tgt_v)
        for i in range(rpw // _L):
            t16 = tgt_v[pl.ds(i * _L, _L)]
            row16 = base + i * _L + lax.iota(jnp.int32, _L)
            addr = ((row16 >> 3) * (8 * N_CLS) + (t16 >> 7) * 1024
                    + (row16 & 7) * 128 + (t16 & 127))
            idx_v[pl.ds(i * _L, _L)] = addr
        pltpu.async_copy(x_flat_hbm.at[idx_v], g_v, sem).wait()
        pltpu.sync_copy(g_v, out_hbm.at[wid])

    return sc_gather


def _tc_body(tgt_ref, g_ref, x_ref, out_ref):
    j = pl.program_id(0)
    x = x_ref[...]
    tgt = tgt_ref[...]
    tmask = tgt != PAD

    rs = jnp.sum(x, axis=1, keepdims=True)
    part_masked = jnp.sum(jnp.where(tmask, rs, 0.0))
    col0_masked = jnp.sum(jnp.where(tmask, x[:, 0:1], 0.0))
    cnt = jnp.sum(tmask.astype(jnp.float32))
    g = g_ref[...]
    gsum = jnp.sum(jnp.where(tmask, g, 0.0))

    part = (C0 * cnt - EPS * (part_masked - col0_masked)
            - (CONF - EPS) * gsum)

    @pl.when(j == 0)
    def _init():
        out_ref[0, 0] = part

    @pl.when(j != 0)
    def _acc():
        out_ref[0, 0] += part


def kernel(x, target):
    n, c = x.shape
    x_tiled_view = (x.reshape(n // 8, 8, c // 128, 128)
                    .transpose(0, 2, 1, 3).reshape(-1))
    g = _sc_gather_build(n)(x_tiled_view, target)
    out = pl.pallas_call(
        _tc_body,
        grid=(n // RBLK,),
        in_specs=[
            pl.BlockSpec((RBLK, 1), lambda j: (j, 0)),
            pl.BlockSpec((RBLK, 1), lambda j: (j, 0)),
            pl.BlockSpec((RBLK, c), lambda j: (j, 0)),
        ],
        out_specs=pl.BlockSpec((1, 1), lambda j: (0, 0),
                               memory_space=pltpu.SMEM),
        out_shape=jax.ShapeDtypeStruct((1, 1), jnp.float32),
    )(target.reshape(n, 1), g.reshape(n, 1), x)
    return out[0, 0]


# speedup vs baseline: 1.1198x; 1.1198x over previous
import math

import jax
import jax.numpy as jnp
from jax.experimental import pallas as pl
from jax.experimental.pallas import tpu as pltpu

N_CLS = 32000
PAD = 0
EPS = 0.1 / (N_CLS - 2)
CONF = 0.9
C0 = (N_CLS - 2) * EPS * math.log(EPS) + CONF * math.log(CONF)

RBLK = 128


def _body(tgt_ref, x_ref, out_ref):
    j = pl.program_id(0)
    x = x_ref[...]
    tgt = tgt_ref[pl.ds(j * RBLK, RBLK), :]
    tmask = tgt != PAD

    rs = jnp.sum(x, axis=1, keepdims=True)
    part_masked = jnp.sum(jnp.where(tmask, rs, 0.0))
    col = jax.lax.broadcasted_iota(jnp.int32, (1, N_CLS), 1)
    pm = jnp.sum(jnp.where(col == tgt, x, 0.0), axis=1, keepdims=True)
    part_match = jnp.sum(pm)

    col0 = x[:, 0:1]
    col0_all = jnp.sum(col0)
    col0_masked = jnp.sum(jnp.where(tmask, col0, 0.0))
    cnt = jnp.sum(tmask.astype(jnp.float32))

    part = (C0 * cnt
            - EPS * (part_masked - col0_masked)
            - (CONF - EPS) * (part_match - (col0_all - col0_masked)))

    @pl.when(j == 0)
    def _init():
        out_ref[0, 0] = part

    @pl.when(j != 0)
    def _acc():
        out_ref[0, 0] += part


def kernel(x, target):
    n, c = x.shape
    out = pl.pallas_call(
        _body,
        grid=(n // RBLK,),
        in_specs=[
            pl.BlockSpec((n, 1), lambda j: (0, 0)),
            pl.BlockSpec((RBLK, c), lambda j: (j, 0)),
        ],
        out_specs=pl.BlockSpec((1, 1), lambda j: (0, 0),
                               memory_space=pltpu.SMEM),
        out_shape=jax.ShapeDtypeStruct((1, 1), jnp.float32),
    )(target.reshape(n, 1), x)
    return out[0, 0]
